# baseline (speedup 1.0000x reference)
import functools

import jax
import jax.numpy as jnp
from jax import lax
from jax.experimental import pallas as pl
from jax.experimental.pallas import tpu as pltpu
from jax.experimental.pallas import tpu_sc as plsc

K = 5
CLASSES = 1000
BATCH = 16384

def _fixed_ind():
    try:
        cpu = jax.local_devices(backend="cpu")[0]
        with jax.default_device(cpu):
            perm = jax.random.permutation(jax.random.key(42), CLASSES - 1)
            return tuple(int(v) for v in perm[:K])
    except Exception:
        return (955, 914, 121, 753, 617)


_IND = _fixed_ind()

_NC, _NS = 2, 16
_NW = _NC * _NS
_RPW = BATCH // _NW
_VALS = K + 1
_GPW = _RPW * _VALS
_CH = 128
_NCHUNK = _GPW // _CH


def _sc_gather_body(yflat_hbm, y_hbm, out_hbm, y_v, idx_v, gat_v, sem):
    wid = lax.axis_index("s") * _NC + lax.axis_index("c")
    base = wid * _RPW

    pltpu.sync_copy(y_hbm.at[pl.ds(base, _RPW)], y_v)

    lane = lax.iota(jnp.int32, 16)
    for g in range(_RPW // 16):
        y16 = y_v[pl.ds(g * 16, 16)]
        rows1000 = (base + g * 16 + lane) * CLASSES
        idx_v[pl.ds(g * 16, 16)] = rows1000 + y16
        for j in range(K):
            col = jnp.where(y16 <= _IND[j], _IND[j] + 1, _IND[j])
            idx_v[pl.ds((j + 1) * _RPW + g * 16, 16)] = rows1000 + col

    copies = []
    for t in range(_NCHUNK):
        copies.append(
            pltpu.async_copy(
                yflat_hbm.at[idx_v.at[pl.ds(t * _CH, _CH)]],
                gat_v.at[pl.ds(t * _CH, _CH)],
                sem,
            )
        )
    for c in copies:
        c.wait()

    pltpu.sync_copy(gat_v, out_hbm.at[pl.ds(wid * _GPW, _GPW)])


_sc_gather = functools.partial(
    pl.kernel,
    out_type=jax.ShapeDtypeStruct((_NW * _GPW,), jnp.float32),
    mesh=plsc.VectorSubcoreMesh(core_axis_name="c", subcore_axis_name="s"),
    scratch_types=[
        pltpu.VMEM((_RPW,), jnp.int32),
        pltpu.VMEM((_GPW,), jnp.int32),
        pltpu.VMEM((_GPW,), jnp.float32),
        pltpu.SemaphoreType.DMA,
    ],
)(_sc_gather_body)


def _tc_loss_body(g_ref, out_ref):
    x = g_ref[...]
    m = jnp.max(x, axis=1, keepdims=True)
    e = jnp.exp(x - m)
    s = jnp.sum(e, axis=1, keepdims=True)
    p = e / s
    term = jnp.log(p[:, 0, :] + 1e-10) + 0.1 * jnp.sum(
        jnp.log((1.0 - p[:, 1:, :]) + 1e-10), axis=1
    )
    out_ref[...] = jnp.reshape(-jnp.sum(term) / jnp.float32(BATCH), (1, 1))


def kernel(yHat, y):
    yflat = yHat.reshape(-1)
    gathered = _sc_gather(yflat, y)
    g3d = gathered.reshape(_NW, _VALS, _RPW)
    loss = pl.pallas_call(
        _tc_loss_body,
        out_shape=jax.ShapeDtypeStruct((1, 1), jnp.float32),
    )(g3d)
    return loss[0, 0]


# speedup vs baseline: 7.0225x; 7.0225x over previous
import jax
import jax.numpy as jnp
from jax import lax
from jax.experimental import pallas as pl

K = 5
CLASSES = 1000
BATCH = 16384

_BB = 2048
_NBLK = BATCH // _BB


def _fixed_ind():
    try:
        cpu = jax.local_devices(backend="cpu")[0]
        with jax.default_device(cpu):
            perm = jax.random.permutation(jax.random.key(42), CLASSES - 1)
            return tuple(int(v) for v in perm[:K])
    except Exception:
        return (955, 914, 121, 753, 617)


_IND = _fixed_ind()


def _loss_body(x_ref, y_ref, out_ref):
    x = x_ref[...]
    yb = y_ref[0]

    ci = lax.broadcasted_iota(jnp.int32, (CLASSES, _BB), 0)
    tv = jnp.sum(jnp.where(ci == yb, x, 0.0), axis=0, keepdims=True)

    vals = [tv]
    for j in range(K):
        lo = x[_IND[j]:_IND[j] + 1, :]
        hi = x[_IND[j] + 1:_IND[j] + 2, :]
        vals.append(jnp.where(yb <= _IND[j], hi, lo))
    v = jnp.concatenate(vals, axis=0)

    m = jnp.max(v, axis=0, keepdims=True)
    e = jnp.exp(v - m)
    s = jnp.sum(e, axis=0, keepdims=True)
    p = e / s
    term = jnp.log(p[0:1, :] + 1e-10) + 0.1 * jnp.sum(
        jnp.log((1.0 - p[1:, :]) + 1e-10), axis=0, keepdims=True
    )
    part = -jnp.sum(term) / jnp.float32(BATCH)

    @pl.when(pl.program_id(0) == 0)
    def _init():
        out_ref[...] = jnp.zeros((1, 1), jnp.float32)

    out_ref[...] += jnp.reshape(part, (1, 1))


def kernel(yHat, y):
    xT = yHat.T
    y3 = y.reshape(_NBLK, 1, _BB)
    loss = pl.pallas_call(
        _loss_body,
        grid=(_NBLK,),
        in_specs=[
            pl.BlockSpec((CLASSES, _BB), lambda b: (0, b)),
            pl.BlockSpec((1, 1, _BB), lambda b: (b, 0, 0)),
        ],
        out_specs=pl.BlockSpec((1, 1), lambda b: (0, 0)),
        out_shape=jax.ShapeDtypeStruct((1, 1), jnp.float32),
    )(xT, y3)
    return loss[0, 0]
